# baseline (speedup 1.0000x reference)
import functools

import jax
import jax.numpy as jnp
import numpy as np
from jax import lax
from jax.experimental import pallas as pl
from jax.experimental.pallas import tpu as pltpu
from jax.experimental.pallas import tpu_sc as plsc

_LEVEL0_CODES = [0, 1, 1, 1, 1, 2, 2, 2, 2, 3, 3, 3, 3, 3, 4, 4, 4, 4, 5, 6, 7, 7, 7]
_INTRA_DIST = [0, 0, 1, 2, 3, 0, 1, 2, 3, 0, 1, 2, 3, 4, 0, 1, 2, 3, 0, 0, 0, 1, 2]
_NCLS = 23
_NROWS = 24
_E = 32

_ONEHOT = np.zeros((_NROWS, 8), dtype=np.float32)
for _c, _m in enumerate(_LEVEL0_CODES):
    _ONEHOT[_c, _m] = 1.0
_DIST = np.zeros((_NROWS, 1), dtype=np.float32)
_DIST[:_NCLS, 0] = np.asarray(_INTRA_DIST, dtype=np.float32)


def _table_body(w_ref, b_ref, onehot_ref, dist_ref, out_ref):
    onehot = onehot_ref[...]
    dist = dist_ref[...]
    w = w_ref[...]
    b = b_ref[...]
    acc = jnp.zeros((_NROWS, _E), dtype=jnp.float32)
    for m in range(8):
        sel = onehot[:, m : m + 1]
        acc = acc + sel * (b[m : m + 1, :] + dist * w[m : m + 1, :])
    out_ref[...] = acc


def _build_table(weights, bias):
    return pl.pallas_call(
        _table_body,
        out_shape=jax.ShapeDtypeStruct((_NROWS, _E), jnp.float32),
    )(weights, bias, jnp.asarray(_ONEHOT), jnp.asarray(_DIST))


_INFO = plsc.get_sparse_core_info()
_NC = _INFO.num_cores
_NS = _INFO.num_subcores
_NW = _NC * _NS

_B = 128
_T = 32
_HW = 256
_L = 16
_NBUF = 4
_WCHUNK = 4
_NCHUNK = _HW // _WCHUNK


@functools.partial(
    pl.kernel,
    mesh=plsc.VectorSubcoreMesh(core_axis_name="c", subcore_axis_name="s"),
    compiler_params=pltpu.CompilerParams(
        use_tc_tiling_on_sc=False, needs_layout_passes=False
    ),
    out_type=jax.ShapeDtypeStruct((_T * _HW * _E * _B,), jnp.float32),
    scratch_types=[
        pltpu.VMEM((_NROWS * _E,), jnp.float32),
        pltpu.VMEM((_NROWS * _E * _L,), jnp.float32),
        pltpu.VMEM((_HW * _B,), jnp.int32),
        [pltpu.VMEM((_WCHUNK * _E * _B,), jnp.float32) for _ in range(_NBUF)],
        pltpu.SemaphoreType.DMA,
        [pltpu.SemaphoreType.DMA for _ in range(_NBUF)],
    ],
)
def _sc_gather(table_hbm, idx_hbm, out_hbm, table_v, rep_v, idx_v, rows, isem, osems):
    wid = lax.axis_index("s") * _NC + lax.axis_index("c")

    idx_cp = pltpu.async_copy(
        idx_hbm.at[pl.ds(wid * (_HW * _B), _HW * _B)], idx_v, isem
    )

    pltpu.sync_copy(table_hbm, table_v)

    @plsc.parallel_loop(0, _NROWS * _E // _L, 1)
    def _(j):
        v = table_v[pl.ds(j * _L, _L)]
        for l in range(_L):
            src = j * _L + l
            c = src >> 5
            e = src & 31
            rep_v[pl.ds((e * _NROWS + c) * _L, _L)] = (
                jnp.full((_L,), 0, jnp.float32) + v[l]
            )

    idx_cp.wait()

    def out_slice(j):
        return out_hbm.at[pl.ds((wid * _HW + j * _WCHUNK) * (_E * _B), _WCHUNK * _E * _B)]

    def step(j, carry):
        for k in range(_NBUF):
            cid = j * _NBUF + k

            @pl.when(j > 0)
            def _(k=k, cid=cid):
                pltpu.make_async_copy(rows[k], out_slice(cid - _NBUF), osems[k]).wait()

            rows_k = rows[k]
            c0 = cid * _WCHUNK

            iota = lax.broadcasted_iota(jnp.int32, (_L,), 0)

            @plsc.parallel_loop(0, _WCHUNK * 8, 1)
            def _(i):
                ul = i >> 3
                kb = i & 7
                cls = idx_v[pl.ds((c0 + ul) * _B + kb * _L, _L)]
                clsbase = (cls << 4) + iota

                obase = ul * (_E * _B) + kb * _L

                for e in range(_E):
                    val = plsc.load_gather(rep_v, [clsbase + e * (_NROWS * _L)])
                    rows_k[pl.ds(obase + e * _B, _L)] = val

            pltpu.async_copy(rows_k, out_slice(cid), osems[k])
        return carry

    lax.fori_loop(0, _NCHUNK // _NBUF, step, 0)

    for k in range(_NBUF):
        last = (_NCHUNK // _NBUF - 1) * _NBUF + k
        pltpu.make_async_copy(rows[k], out_slice(last), osems[k]).wait()


def kernel(input, weights, bias):
    table = _build_table(weights, bias)
    idx = jnp.transpose(input.reshape(_B, _T, _HW), (1, 2, 0)).reshape(-1)
    out = _sc_gather(table.reshape(-1), idx).reshape(_T, 16, 16, _E, _B)
    return jnp.transpose(out, (4, 0, 1, 2, 3))


# speedup vs baseline: 1.0122x; 1.0122x over previous
import functools

import jax
import jax.numpy as jnp
import numpy as np
from jax import lax
from jax.experimental import pallas as pl
from jax.experimental.pallas import tpu as pltpu
from jax.experimental.pallas import tpu_sc as plsc

_LEVEL0_CODES = [0, 1, 1, 1, 1, 2, 2, 2, 2, 3, 3, 3, 3, 3, 4, 4, 4, 4, 5, 6, 7, 7, 7]
_INTRA_DIST = [0, 0, 1, 2, 3, 0, 1, 2, 3, 0, 1, 2, 3, 4, 0, 1, 2, 3, 0, 0, 0, 1, 2]
_NCLS = 23
_NROWS = 24
_E = 32

_ONEHOT = np.zeros((_NROWS, 8), dtype=np.float32)
for _c, _m in enumerate(_LEVEL0_CODES):
    _ONEHOT[_c, _m] = 1.0
_DIST = np.zeros((_NROWS, 1), dtype=np.float32)
_DIST[:_NCLS, 0] = np.asarray(_INTRA_DIST, dtype=np.float32)


def _table_body(w_ref, b_ref, onehot_ref, dist_ref, out_ref):
    onehot = onehot_ref[...]
    dist = dist_ref[...]
    w = w_ref[...]
    b = b_ref[...]
    acc = jnp.zeros((_NROWS, _E), dtype=jnp.float32)
    for m in range(8):
        sel = onehot[:, m : m + 1]
        acc = acc + sel * (b[m : m + 1, :] + dist * w[m : m + 1, :])
    out_ref[...] = acc


def _build_table(weights, bias):
    return pl.pallas_call(
        _table_body,
        out_shape=jax.ShapeDtypeStruct((_NROWS, _E), jnp.float32),
    )(weights, bias, jnp.asarray(_ONEHOT), jnp.asarray(_DIST))


_INFO = plsc.get_sparse_core_info()
_NC = _INFO.num_cores
_NS = _INFO.num_subcores
_NW = _NC * _NS

_B = 128
_T = 32
_HW = 256
_L = 16
_NBUF = 2
_WCHUNK = 4
_NCHUNK = _HW // _WCHUNK


@functools.partial(
    pl.kernel,
    mesh=plsc.VectorSubcoreMesh(core_axis_name="c", subcore_axis_name="s"),
    compiler_params=pltpu.CompilerParams(
        use_tc_tiling_on_sc=False, needs_layout_passes=False
    ),
    out_type=jax.ShapeDtypeStruct((_T * _HW * _E * _B,), jnp.float32),
    scratch_types=[
        pltpu.VMEM((_NROWS * _E,), jnp.float32),
        pltpu.VMEM((_NROWS * _E * _L,), jnp.float32),
        pltpu.VMEM((_HW * _B,), jnp.int32),
        [pltpu.VMEM((_WCHUNK * _E * _B,), jnp.float32) for _ in range(_NBUF)],
        pltpu.SemaphoreType.DMA,
        [pltpu.SemaphoreType.DMA for _ in range(_NBUF)],
    ],
)
def _sc_gather(table_hbm, idx_hbm, out_hbm, table_v, rep_v, idx_v, rows, isem, osems):
    wid = lax.axis_index("s") * _NC + lax.axis_index("c")

    idx_cp = pltpu.async_copy(
        idx_hbm.at[pl.ds(wid * (_HW * _B), _HW * _B)], idx_v, isem
    )

    pltpu.sync_copy(table_hbm, table_v)

    @plsc.parallel_loop(0, _NROWS * _E // _L, 1)
    def _(j):
        v = table_v[pl.ds(j * _L, _L)]
        for l in range(_L):
            src = j * _L + l
            c = src >> 5
            e = src & 31
            rep_v[pl.ds((e * _NROWS + c) * _L, _L)] = (
                jnp.full((_L,), 0, jnp.float32) + v[l]
            )

    idx_cp.wait()

    def out_slice(j):
        return out_hbm.at[pl.ds((wid * _HW + j * _WCHUNK) * (_E * _B), _WCHUNK * _E * _B)]

    def step(j, carry):
        for k in range(_NBUF):
            cid = j * _NBUF + k

            @pl.when(j > 0)
            def _(k=k, cid=cid):
                pltpu.make_async_copy(rows[k], out_slice(cid - _NBUF), osems[k]).wait()

            rows_k = rows[k]
            c0 = cid * _WCHUNK

            iota = lax.broadcasted_iota(jnp.int32, (_L,), 0)

            @plsc.parallel_loop(0, _WCHUNK * 8, 1)
            def _(i):
                ul = i >> 3
                kb = i & 7
                cls = idx_v[pl.ds((c0 + ul) * _B + kb * _L, _L)]
                clsbase = (cls << 4) + iota

                obase = ul * (_E * _B) + kb * _L

                for e in range(_E):
                    val = plsc.load_gather(rep_v, [clsbase + e * (_NROWS * _L)])
                    rows_k[pl.ds(obase + e * _B, _L)] = val

            pltpu.async_copy(rows_k, out_slice(cid), osems[k])
        return carry

    lax.fori_loop(0, _NCHUNK // _NBUF, step, 0)

    for k in range(_NBUF):
        last = (_NCHUNK // _NBUF - 1) * _NBUF + k
        pltpu.make_async_copy(rows[k], out_slice(last), osems[k]).wait()


def kernel(input, weights, bias):
    table = _build_table(weights, bias)
    idx = jnp.transpose(input.reshape(_B, _T, _HW), (1, 2, 0)).reshape(-1)
    out = _sc_gather(table.reshape(-1), idx).reshape(_T, 16, 16, _E, _B)
    return jnp.transpose(out, (4, 0, 1, 2, 3))
